# baseline (speedup 1.0000x reference)
import functools

import jax
import jax.numpy as jnp
from jax import lax
from jax.experimental import pallas as pl
from jax.experimental.pallas import tpu as pltpu
from jax.experimental.pallas import tpu_sc as plsc

K = 1024
C = 64
HW = 1024
N = 16 * HW


def _vq_idx_kernel(xb_ref, cbt2_ref, aug_ref, idx_ref):
    flat = xb_ref[0].reshape(HW, C)
    cbt2 = cbt2_ref[...]
    aug = aug_ref[...]
    cb_sqr = cbsq_ref = None

    mm2 = lax.dot_general(flat.astype(jnp.bfloat16), cbt2,
                          (((1,), (0,)), ((), ())),
                          preferred_element_type=jnp.float32)
    x_sqr = jnp.sum(flat * flat, axis=1, keepdims=True)
    cb_sqr = jnp.sum((-0.5 * cbt2) * (-0.5 * cbt2), axis=0,
                     keepdims=True)
    dist = (x_sqr + cb_sqr) + mm2

    minval = jnp.min(dist, axis=1, keepdims=True)
    eq = dist == minval
    onehot = jnp.where(eq, 1.0, 0.0)
    agg = lax.dot_general(onehot, aug, (((1,), (1,)), ((), ())),
                          preferred_element_type=jnp.float32)
    idx_f = agg[:, 0:1] + agg[:, 1:2]
    idx_ref[0] = idx_f.astype(jnp.int32).reshape(32, 32)

    has_tie = jnp.max(agg[:, 2:3]) > 1.5

    @pl.when(has_tie)
    def _fix_ties():
        iota_f = lax.broadcasted_iota(jnp.int32, (HW, K), 1).astype(jnp.float32)
        masked = jnp.where(eq, iota_f, jnp.float32(K))
        idx2 = jnp.min(masked, axis=1, keepdims=True)
        idx_ref[0] = idx2.astype(jnp.int32).reshape(32, 32)


def _make_sc_gather():
    info = plsc.get_sparse_core_info()
    nw = info.num_cores * info.num_subcores
    bpw = N // nw
    mesh = plsc.VectorSubcoreMesh(core_axis_name="c", subcore_axis_name="s")

    @functools.partial(
        pl.kernel, mesh=mesh,
        out_type=jax.ShapeDtypeStruct((N, 128), jnp.float32),
        scratch_types=[
            pltpu.VMEM((bpw,), jnp.int32),
            pltpu.VMEM((bpw, 128), jnp.float32),
            pltpu.SemaphoreType.DMA,
        ],
    )
    def gather_k(table_hbm, idx_hbm, out_hbm, idx_v, rows_v, sem):
        wid = lax.axis_index("s") * info.num_cores + lax.axis_index("c")
        base = wid * bpw
        pltpu.sync_copy(idx_hbm.at[pl.ds(base, bpw)], idx_v)
        pltpu.async_copy(table_hbm.at[idx_v], rows_v, sem).wait()
        pltpu.sync_copy(rows_v, out_hbm.at[pl.ds(base, bpw)])

    return gather_k


@jax.jit
def kernel(x, codebook):
    B = x.shape[0]
    xt = jnp.transpose(x, (0, 2, 3, 1))
    cbt2 = -2.0 * codebook.T
    ks = jnp.arange(K, dtype=jnp.int32)
    aug = jnp.concatenate(
        [(ks & ~255).astype(jnp.float32)[None, :],
         (ks & 255).astype(jnp.float32)[None, :],
         jnp.ones((1, K), jnp.float32),
         jnp.zeros((5, K), jnp.float32)], axis=0)
    ind_out = pl.pallas_call(
        _vq_idx_kernel,
        grid=(B,),
        in_specs=[
            pl.BlockSpec((1, 32, 32, C), lambda b: (b, 0, 0, 0)),
            pl.BlockSpec((C, K), lambda b: (0, 0)),
            pl.BlockSpec((8, K), lambda b: (0, 0)),
        ],
        out_specs=pl.BlockSpec((1, 32, 32), lambda b: (b, 0, 0)),
        out_shape=jax.ShapeDtypeStruct((B, 32, 32), jnp.int32),
    )(xt, cbt2, aug)

    ind_flat = ind_out.reshape(N)
    table128 = jnp.pad(codebook, ((0, 0), (0, 128 - C)))
    codes128 = _make_sc_gather()(table128, ind_flat)
    codes_bhwc = codes128.reshape(B, 32, 32, 128)[..., :C]
    codes_out = jnp.transpose(codes_bhwc, (0, 3, 1, 2))
    return codes_out, ind_out


# speedup vs baseline: 2.7058x; 2.7058x over previous
import jax
import jax.numpy as jnp
from jax import lax
from jax.experimental import pallas as pl

K = 1024
C = 64
HW = 1024


def _vq_kernel(xb_ref, cbt2_ref, aug_ref, codes_ref, idx_ref):
    flat = xb_ref[...].reshape(2 * HW, C)
    cbt2 = cbt2_ref[...]
    aug = aug_ref[...]
    cbt = aug[:C, :]

    mm2 = lax.dot_general(flat.astype(jnp.bfloat16), cbt2,
                          (((1,), (0,)), ((), ())),
                          preferred_element_type=jnp.float32)
    x_sqr = jnp.sum(flat * flat, axis=1, keepdims=True)
    cb_sqr = jnp.sum(cbt * cbt, axis=0, keepdims=True)
    dist = (x_sqr + cb_sqr) + mm2

    minval = jnp.min(dist, axis=1, keepdims=True)
    eq = dist == minval
    onehot = jnp.where(eq, 1.0, 0.0)
    agg = lax.dot_general(onehot, aug, (((1,), (1,)), ((), ())),
                          preferred_element_type=jnp.float32)
    codes_ref[...] = agg[:, :C].reshape(2, 32, 32, C)
    idx_f = agg[:, C:C + 1] + agg[:, C + 1:C + 2]
    idx_ref[...] = idx_f.astype(jnp.int32).reshape(2, 32, 32)

    nties = agg[:, C + 2:C + 3]
    has_tie = jnp.max(nties) > 1.5

    @pl.when(has_tie)
    def _fix_ties():
        iota_f = lax.broadcasted_iota(jnp.int32, (2 * HW, K), 1).astype(jnp.float32)
        masked = jnp.where(eq, iota_f, jnp.float32(K))
        idx2 = jnp.min(masked, axis=1, keepdims=True)
        onehot2 = jnp.where(masked == idx2, 1.0, 0.0)
        agg2 = lax.dot_general(onehot2, aug, (((1,), (1,)), ((), ())),
                               preferred_element_type=jnp.float32)
        codes_ref[...] = agg2[:, :C].reshape(2, 32, 32, C)
        idx_ref[...] = idx2.astype(jnp.int32).reshape(2, 32, 32)


@jax.jit
def kernel(x, codebook):
    B = x.shape[0]
    xt = jnp.transpose(x, (0, 2, 3, 1))
    cbt = codebook.T
    cbt2 = -2.0 * cbt
    ks = jnp.arange(K, dtype=jnp.int32)
    aug = jnp.concatenate(
        [cbt,
         (ks & ~255).astype(jnp.float32)[None, :],
         (ks & 255).astype(jnp.float32)[None, :],
         jnp.ones((1, K), jnp.float32)], axis=0)
    codes_bhwc, ind_out = pl.pallas_call(
        _vq_kernel,
        grid=(B // 2,),
        in_specs=[
            pl.BlockSpec((2, 32, 32, C), lambda b: (b, 0, 0, 0)),
            pl.BlockSpec((C, K), lambda b: (0, 0)),
            pl.BlockSpec((C + 3, K), lambda b: (0, 0)),
        ],
        out_specs=[
            pl.BlockSpec((2, 32, 32, C), lambda b: (b, 0, 0, 0)),
            pl.BlockSpec((2, 32, 32), lambda b: (b, 0, 0)),
        ],
        out_shape=[
            jax.ShapeDtypeStruct((B, 32, 32, C), jnp.float32),
            jax.ShapeDtypeStruct((B, 32, 32), jnp.int32),
        ],
    )(xt, cbt2, aug)
    codes_out = jnp.transpose(codes_bhwc, (0, 3, 1, 2))
    return codes_out, ind_out


# speedup vs baseline: 2.8124x; 1.0394x over previous
import jax
import jax.numpy as jnp
from jax import lax
from jax.experimental import pallas as pl

K = 1024
C = 64
HW = 1024


def _vq_kernel(xb_ref, cbt2_ref, aug_ref, codes_ref, idx_ref):
    flat = xb_ref[...].reshape(4 * HW, C)
    cbt2 = cbt2_ref[...]
    aug = aug_ref[...]
    cbt = aug[:C, :]

    mm2 = lax.dot_general(flat.astype(jnp.bfloat16), cbt2,
                          (((1,), (0,)), ((), ())),
                          preferred_element_type=jnp.float32)
    x_sqr = jnp.sum(flat * flat, axis=1, keepdims=True)
    cb_sqr = jnp.sum(cbt * cbt, axis=0, keepdims=True)
    dist = (x_sqr + cb_sqr) + mm2

    minval = jnp.min(dist, axis=1, keepdims=True)
    eq = dist == minval
    onehot = jnp.where(eq, 1.0, 0.0)
    agg = lax.dot_general(onehot, aug, (((1,), (1,)), ((), ())),
                          preferred_element_type=jnp.float32)
    codes_ref[...] = agg[:, :C].reshape(4, 32, 32, C)
    idx_f = agg[:, C:C + 1] + agg[:, C + 1:C + 2]
    idx_ref[...] = idx_f.astype(jnp.int32).reshape(4, 32, 32)

    nties = agg[:, C + 2:C + 3]
    has_tie = jnp.max(nties) > 1.5

    @pl.when(has_tie)
    def _fix_ties():
        iota_f = lax.broadcasted_iota(jnp.int32, (4 * HW, K), 1).astype(jnp.float32)
        masked = jnp.where(eq, iota_f, jnp.float32(K))
        idx2 = jnp.min(masked, axis=1, keepdims=True)
        onehot2 = jnp.where(masked == idx2, 1.0, 0.0)
        agg2 = lax.dot_general(onehot2, aug, (((1,), (1,)), ((), ())),
                               preferred_element_type=jnp.float32)
        codes_ref[...] = agg2[:, :C].reshape(4, 32, 32, C)
        idx_ref[...] = idx2.astype(jnp.int32).reshape(4, 32, 32)


@jax.jit
def kernel(x, codebook):
    B = x.shape[0]
    xt = jnp.transpose(x, (0, 2, 3, 1))
    cbt = codebook.T
    cbt2 = -2.0 * cbt
    ks = jnp.arange(K, dtype=jnp.int32)
    aug = jnp.concatenate(
        [cbt,
         (ks & ~255).astype(jnp.float32)[None, :],
         (ks & 255).astype(jnp.float32)[None, :],
         jnp.ones((1, K), jnp.float32)], axis=0)
    codes_bhwc, ind_out = pl.pallas_call(
        _vq_kernel,
        grid=(B // 4,),
        in_specs=[
            pl.BlockSpec((4, 32, 32, C), lambda b: (b, 0, 0, 0)),
            pl.BlockSpec((C, K), lambda b: (0, 0)),
            pl.BlockSpec((C + 3, K), lambda b: (0, 0)),
        ],
        out_specs=[
            pl.BlockSpec((4, 32, 32, C), lambda b: (b, 0, 0, 0)),
            pl.BlockSpec((4, 32, 32), lambda b: (b, 0, 0)),
        ],
        out_shape=[
            jax.ShapeDtypeStruct((B, 32, 32, C), jnp.float32),
            jax.ShapeDtypeStruct((B, 32, 32), jnp.int32),
        ],
    )(xt, cbt2, aug)
    codes_out = jnp.transpose(codes_bhwc, (0, 3, 1, 2))
    return codes_out, ind_out
